# baseline (speedup 1.0000x reference)
import jax
import jax.numpy as jnp
from jax import lax
from jax.experimental import pallas as pl
from jax.experimental.pallas import tpu as pltpu
from jax.experimental.pallas import tpu_sc as plsc

_K = 16
_L = 16
_NW = 32
_QPW = 32
_GV = 32


def _sc_body(pts_hbm, q_hbm, dens_hbm, out_hbm, pts_v, q_v, ob_v, d2_v, sem):
    fp = pts_hbm.shape[1] // 3
    p = fp // 4
    steps = p // _L
    wid = lax.axis_index("s") * 2 + lax.axis_index("c")
    b = wid // (_NW // pts_hbm.shape[0])

    pltpu.sync_copy(pts_hbm.at[b], pts_v)
    pltpu.sync_copy(q_hbm.at[wid], q_v)

    lanes = lax.iota(jnp.int32, _L)
    inf = jnp.float32(jnp.inf)

    def _splat(slot):
        return plsc.load_gather(q_v, [jnp.full((_L,), slot, jnp.int32)])

    def per_query(i, carry):
        qx = _splat(i)
        qy = _splat(_QPW + i)
        qz = _splat(2 * _QPW + i)
        qff = _splat(4 * _QPW + i)
        qfi = qff.astype(jnp.int32) - 1
        basev = qfi * p + lanes

        def group_step(g, tk):
            td, ti, thr = tk
            goff = g * (_GV * _L)

            def scan_v(j, gm):
                idxv = basev + goff + j * _L
                xv = plsc.load_gather(pts_v, [idxv])
                yv = plsc.load_gather(pts_v, [idxv + fp])
                zv = plsc.load_gather(pts_v, [idxv + 2 * fp])
                dx = xv - qx
                dy = yv - qy
                dz = zv - qz
                d2 = dx * dx + dy * dy + dz * dz
                d2_v[pl.ds(j * _L, _L)] = d2
                return jnp.minimum(gm, d2)

            gmin = lax.fori_loop(0, _GV, scan_v,
                                 jnp.full((_L,), inf, jnp.float32))

            def process(td, ti, thr):
                def merge_v(j, tk2):
                    td, ti, thr = tk2
                    d2 = d2_v[pl.ds(j * _L, _L)]
                    idxv = basev + goff + j * _L

                    def merge(td, ti, thr):
                        cd, ci = plsc.sort_key_val(d2, idxv)
                        rd = lax.rev(td, (0,))
                        ri = lax.rev(ti, (0,))
                        take = (cd < rd) | ((cd == rd) & (ci < ri))
                        nd = jnp.where(take, cd, rd)
                        ni = jnp.where(take, ci, ri)
                        nd, ni = plsc.sort_key_val(nd, ni)
                        return nd, ni, jnp.broadcast_to(nd[_L - 1], (_L,))

                    def keep(td, ti, thr):
                        return td, ti, thr

                    return lax.cond(jnp.any(d2 < thr), merge, keep,
                                    td, ti, thr)

                return lax.fori_loop(0, _GV, merge_v, (td, ti, thr))

            def skip(td, ti, thr):
                return td, ti, thr

            return lax.cond(jnp.any(gmin < thr), process, skip, td, ti, thr)

        td0 = jnp.full((_L,), inf, jnp.float32)
        ti0 = jnp.zeros((_L,), jnp.int32)
        td, ti, _ = lax.fori_loop(0, steps // _GV, group_step,
                                  (td0, ti0, td0))

        xg = plsc.load_gather(pts_v, [ti])
        yg = plsc.load_gather(pts_v, [ti + fp])
        zg = plsc.load_gather(pts_v, [ti + 2 * fp])
        row = i * (5 * _K)
        ob_v[pl.ds(row, _K)] = xg - qx
        ob_v[pl.ds(row + _K, _K)] = yg - qy
        ob_v[pl.ds(row + 2 * _K, _K)] = zg - qz
        ob_v[pl.ds(row + 4 * _K, _K)] = qff - 1.0
        didx = ti + b * fp
        pltpu.async_copy(dens_hbm.at[didx],
                         ob_v.at[pl.ds(row + 3 * _K, _K)], sem).wait()
        return carry

    lax.fori_loop(0, _QPW, per_query, 0)
    pltpu.sync_copy(ob_v, out_hbm.at[pl.ds(wid * (_QPW * 5 * _K),
                                           _QPW * 5 * _K)])


def kernel(sample_points, points_frames, K):
    del K
    b, n, _ = sample_points.shape
    _, f, p, c = points_frames.shape
    fp = f * p
    pts = points_frames.reshape(b, fp, c)
    pts_xyz = pts[..., :3].transpose(0, 2, 1).reshape(b, 3 * fp)
    dens = pts[..., 3].reshape(b * fp)
    q = sample_points.reshape(b * n, 5).T
    qg = (q.reshape(5, _NW, _QPW).transpose(1, 0, 2)
          .reshape(_NW, 5 * _QPW))

    out = pl.kernel(
        _sc_body,
        out_type=jax.ShapeDtypeStruct((b * n * 5 * _K,), jnp.float32),
        mesh=plsc.VectorSubcoreMesh(core_axis_name="c", subcore_axis_name="s"),
        compiler_params=pltpu.CompilerParams(needs_layout_passes=False),
        scratch_types=[
            pltpu.VMEM((3 * fp,), jnp.float32),
            pltpu.VMEM((5 * _QPW,), jnp.float32),
            pltpu.VMEM((_QPW * 5 * _K,), jnp.float32),
            pltpu.VMEM((_GV * _L,), jnp.float32),
            pltpu.SemaphoreType.DMA,
        ],
    )(pts_xyz, qg, dens)
    return out.reshape(b, n, 5, _K).transpose(0, 1, 3, 2)


# speedup vs baseline: 7.4326x; 7.4326x over previous
import jax
import jax.numpy as jnp
from jax import lax
from jax.experimental import pallas as pl
from jax.experimental.pallas import tpu as pltpu
from jax.experimental.pallas import tpu_sc as plsc

_K = 16
_L = 16
_NW = 32
_QPW = 32
_C = 8


def _merge_desc(ad, ai, bd, bi):
    take = (bd < ad) | ((bd == ad) & (bi < ai))
    nd = jnp.where(take, bd, ad)
    ni = jnp.where(take, bi, ai)
    return plsc.sort_key_val(nd, ni, descending=True)


def _sc_body(pts_hbm, q_hbm, dens_hbm, out_hbm, pts_v, q_v, ob_v, dr_v, sem):
    fp = pts_hbm.shape[1] // 3
    p = fp // 4
    steps = p // (_L * _C)
    wid = lax.axis_index("s") * 2 + lax.axis_index("c")
    b = wid // (_NW // pts_hbm.shape[0])

    pltpu.sync_copy(pts_hbm.at[b], pts_v)
    pltpu.sync_copy(q_hbm.at[wid], q_v)

    lanes = lax.iota(jnp.int32, _L)
    inf = jnp.float32(jnp.inf)

    def _splat(slot):
        return plsc.load_gather(q_v, [jnp.full((_L,), slot, jnp.int32)])

    def per_query(i, carry):
        qx = _splat(i)
        qy = _splat(_QPW + i)
        qz = _splat(2 * _QPW + i)
        qff = _splat(4 * _QPW + i)
        qfi = qff.astype(jnp.int32) - 1
        basev = qfi * p + lanes

        def scan_step(j, tk):
            out = []
            for ch in range(_C):
                td, ti = tk[2 * ch], tk[2 * ch + 1]
                idxv = basev + (j * _C + ch) * _L
                xv = plsc.load_gather(pts_v, [idxv])
                yv = plsc.load_gather(pts_v, [idxv + fp])
                zv = plsc.load_gather(pts_v, [idxv + 2 * fp])
                dx = xv - qx
                dy = yv - qy
                dz = zv - qz
                d2 = dx * dx + dy * dy + dz * dz
                cd, ci = plsc.sort_key_val(d2, idxv)
                out.extend(_merge_desc(td, ti, cd, ci))
            return tuple(out)

        td0 = jnp.full((_L,), inf, jnp.float32)
        ti0 = jnp.zeros((_L,), jnp.int32)
        tk = lax.fori_loop(0, steps, scan_step, (td0, ti0) * _C)

        pairs = [(tk[2 * ch], tk[2 * ch + 1]) for ch in range(_C)]
        while len(pairs) > 1:
            nxt = []
            for a in range(0, len(pairs), 2):
                (ad, ai), (bd, bi) = pairs[a], pairs[a + 1]
                bd = lax.rev(bd, (0,))
                bi = lax.rev(bi, (0,))
                nxt.append(_merge_desc(ad, ai, bd, bi))
            pairs = nxt
        td, ti = pairs[0]
        td = lax.rev(td, (0,))
        ti = lax.rev(ti, (0,))

        xg = plsc.load_gather(pts_v, [ti])
        yg = plsc.load_gather(pts_v, [ti + fp])
        zg = plsc.load_gather(pts_v, [ti + 2 * fp])
        row = i * (5 * _K)
        ob_v[pl.ds(row, _K)] = xg - qx
        ob_v[pl.ds(row + _K, _K)] = yg - qy
        ob_v[pl.ds(row + 2 * _K, _K)] = zg - qz
        ob_v[pl.ds(row + 4 * _K, _K)] = qff - 1.0
        didx = ti + b * fp
        pltpu.async_copy(dens_hbm.at[didx],
                         ob_v.at[pl.ds(row + 3 * _K, _K)], sem)
        return carry

    lax.fori_loop(0, _QPW, per_query, 0)
    pltpu.make_async_copy(dens_hbm.at[pl.ds(0, _QPW * _K)], dr_v, sem).wait()
    pltpu.sync_copy(ob_v, out_hbm.at[pl.ds(wid * (_QPW * 5 * _K),
                                           _QPW * 5 * _K)])


def kernel(sample_points, points_frames, K):
    del K
    b, n, _ = sample_points.shape
    _, f, p, c = points_frames.shape
    fp = f * p
    pts = points_frames.reshape(b, fp, c)
    pts_xyz = pts[..., :3].transpose(0, 2, 1).reshape(b, 3 * fp)
    dens = pts[..., 3].reshape(b * fp)
    q = sample_points.reshape(b * n, 5).T
    qg = (q.reshape(5, _NW, _QPW).transpose(1, 0, 2)
          .reshape(_NW, 5 * _QPW))

    out = pl.kernel(
        _sc_body,
        out_type=jax.ShapeDtypeStruct((b * n * 5 * _K,), jnp.float32),
        mesh=plsc.VectorSubcoreMesh(core_axis_name="c", subcore_axis_name="s"),
        compiler_params=pltpu.CompilerParams(needs_layout_passes=False),
        scratch_types=[
            pltpu.VMEM((3 * fp,), jnp.float32),
            pltpu.VMEM((5 * _QPW,), jnp.float32),
            pltpu.VMEM((_QPW * 5 * _K,), jnp.float32),
            pltpu.VMEM((_QPW * _K,), jnp.float32),
            pltpu.SemaphoreType.DMA,
        ],
    )(pts_xyz, qg, dens)
    return out.reshape(b, n, 5, _K).transpose(0, 1, 3, 2)


# speedup vs baseline: 8.8197x; 1.1866x over previous
import jax
import jax.numpy as jnp
from jax import lax
from jax.experimental import pallas as pl
from jax.experimental.pallas import tpu as pltpu
from jax.experimental.pallas import tpu_sc as plsc

_K = 16
_L = 16
_NW = 32
_QPW = 32
_C = 8


def _merge_desc(ad, ai, bd, bi):
    take = bd < ad
    nd = jnp.where(take, bd, ad)
    ni = jnp.where(take, bi, ai)
    return plsc.sort_key_val(nd, ni, descending=True)


def _sc_body(xs_hbm, ys_hbm, zs_hbm, q_hbm, dens_hbm, out_hbm, x_v, y_v, z_v, q_v, ob_v, dr_v, sem):
    fp = xs_hbm.shape[1]
    p = fp // 4
    steps = p // (_L * _C)
    wid = lax.axis_index("s") * 2 + lax.axis_index("c")
    b = wid // (_NW // xs_hbm.shape[0])

    pltpu.sync_copy(xs_hbm.at[b], x_v)
    pltpu.sync_copy(ys_hbm.at[b], y_v)
    pltpu.sync_copy(zs_hbm.at[b], z_v)
    pltpu.sync_copy(q_hbm.at[wid], q_v)

    lanes = lax.iota(jnp.int32, _L)
    inf = jnp.float32(jnp.inf)

    def _splat(slot):
        return plsc.load_gather(q_v, [jnp.full((_L,), slot, jnp.int32)])

    def per_query(i, carry):
        qx = _splat(i)
        qy = _splat(_QPW + i)
        qz = _splat(2 * _QPW + i)
        qff = _splat(4 * _QPW + i)
        qfi = qff.astype(jnp.int32) - 1
        basev = qfi * p + lanes

        def scan_step(j, tk):
            out = []
            for ch in range(_C):
                td, ti = tk[2 * ch], tk[2 * ch + 1]
                idxv = basev + (j * _C + ch) * _L
                xv = plsc.load_gather(x_v, [idxv])
                yv = plsc.load_gather(y_v, [idxv])
                zv = plsc.load_gather(z_v, [idxv])
                dx = xv - qx
                dy = yv - qy
                dz = zv - qz
                d2 = dx * dx + dy * dy + dz * dz
                cd, ci = plsc.sort_key_val(d2, idxv)
                out.extend(_merge_desc(td, ti, cd, ci))
            return tuple(out)

        td0 = jnp.full((_L,), inf, jnp.float32)
        ti0 = jnp.zeros((_L,), jnp.int32)
        tk = lax.fori_loop(0, steps, scan_step, (td0, ti0) * _C)

        pairs = [(tk[2 * ch], tk[2 * ch + 1]) for ch in range(_C)]
        while len(pairs) > 1:
            nxt = []
            for a in range(0, len(pairs), 2):
                (ad, ai), (bd, bi) = pairs[a], pairs[a + 1]
                bd = lax.rev(bd, (0,))
                bi = lax.rev(bi, (0,))
                nxt.append(_merge_desc(ad, ai, bd, bi))
            pairs = nxt
        td, ti = pairs[0]
        td = lax.rev(td, (0,))
        ti = lax.rev(ti, (0,))

        xg = plsc.load_gather(x_v, [ti])
        yg = plsc.load_gather(y_v, [ti])
        zg = plsc.load_gather(z_v, [ti])
        row = i * (5 * _K)
        ob_v[pl.ds(row, _K)] = xg - qx
        ob_v[pl.ds(row + _K, _K)] = yg - qy
        ob_v[pl.ds(row + 2 * _K, _K)] = zg - qz
        ob_v[pl.ds(row + 4 * _K, _K)] = qff - 1.0
        didx = ti + b * fp
        pltpu.async_copy(dens_hbm.at[didx],
                         ob_v.at[pl.ds(row + 3 * _K, _K)], sem)
        return carry

    lax.fori_loop(0, _QPW, per_query, 0)
    pltpu.make_async_copy(dens_hbm.at[pl.ds(0, _QPW * _K)], dr_v, sem).wait()
    pltpu.sync_copy(ob_v, out_hbm.at[pl.ds(wid * (_QPW * 5 * _K),
                                           _QPW * 5 * _K)])


def kernel(sample_points, points_frames, K):
    del K
    b, n, _ = sample_points.shape
    _, f, p, c = points_frames.shape
    fp = f * p
    pts = points_frames.reshape(b, fp, c)
    xs, ys, zs = (pts[..., k].reshape(b, fp) for k in range(3))
    dens = pts[..., 3].reshape(b * fp)
    q = sample_points.reshape(b * n, 5).T
    qg = (q.reshape(5, _NW, _QPW).transpose(1, 0, 2)
          .reshape(_NW, 5 * _QPW))

    out = pl.kernel(
        _sc_body,
        out_type=jax.ShapeDtypeStruct((b * n * 5 * _K,), jnp.float32),
        mesh=plsc.VectorSubcoreMesh(core_axis_name="c", subcore_axis_name="s"),
        compiler_params=pltpu.CompilerParams(needs_layout_passes=False),
        scratch_types=[
            pltpu.VMEM((fp,), jnp.float32),
            pltpu.VMEM((fp,), jnp.float32),
            pltpu.VMEM((fp,), jnp.float32),
            pltpu.VMEM((5 * _QPW,), jnp.float32),
            pltpu.VMEM((_QPW * 5 * _K,), jnp.float32),
            pltpu.VMEM((_QPW * _K,), jnp.float32),
            pltpu.SemaphoreType.DMA,
        ],
    )(xs, ys, zs, qg, dens)
    return out.reshape(b, n, 5, _K).transpose(0, 1, 3, 2)
